# baseline (speedup 1.0000x reference)
import functools

import jax
import jax.numpy as jnp
from jax import lax
from jax.experimental import pallas as pl
from jax.experimental.pallas import tpu as pltpu
from jax.experimental.pallas import tpu_sc as plsc

RADIUS2 = 25.0
BIG = 1e30
L = 16


def _sc_body(pred_hbm, gt_hbm, out_hbm, pbuf, tbuf, table, lrow, mb, orow,
             sh, *, n_p, n_groups):
    c = lax.axis_index("c")
    s = lax.axis_index("s")
    b = 2 * c + s // 8
    sg = s % 8

    pltpu.sync_copy(pred_hbm.at[b], pbuf)
    pltpu.sync_copy(gt_hbm.at[b, sg], tbuf)

    tcv = tbuf[0, :]
    txv = tbuf[1, :]
    tyv = tbuf[2, :]

    lane = lax.broadcasted_iota(jnp.int32, (L,), 0)

    TC_CHUNK = 16
    UNROLL = 2

    def scan_targets(tslice):
        tb = [jnp.full((L,), t, jnp.int32) for t in tslice]
        txb = [txv.at[i].get(mode="promise_in_bounds") for i in tb]
        tyb = [tyv.at[i].get(mode="promise_in_bounds") for i in tb]
        tcb = [tcv.at[i].get(mode="promise_in_bounds") for i in tb]

        def tile_step(i, carry):
            accs = list(carry)
            for u in range(UNROLL):
                base = (UNROLL * i + u) * L
                pxv = pbuf[1, pl.ds(base, L)]
                pyv = pbuf[2, pl.ds(base, L)]
                pcv = pbuf[0, pl.ds(base, L)]
                idxv = jnp.full((L,), base, jnp.int32) + lane
                for k in range(len(tslice)):
                    bst, bix = accs[2 * k], accs[2 * k + 1]
                    dx = pxv - txb[k]
                    dy = pyv - tyb[k]
                    d2 = dx * dx + dy * dy
                    key = jnp.where(pcv == tcb[k], d2, BIG)
                    lt = key < bst
                    accs[2 * k] = jnp.where(lt, key, bst)
                    accs[2 * k + 1] = jnp.where(lt, idxv, bix)
            return tuple(accs)

        init = []
        for _ in tslice:
            init += [jnp.full((L,), BIG, jnp.float32), jnp.zeros((L,), jnp.int32)]
        accs = lax.fori_loop(0, n_p // (L * UNROLL), tile_step, tuple(init))

        outs = []
        for k in range(len(tslice)):
            bst, bix = accs[2 * k], accs[2 * k + 1]
            for step in (8, 4, 2, 1):
                perm = jnp.bitwise_and(lane + step, L - 1)
                pd2 = bst.at[perm].get(mode="promise_in_bounds")
                pix = bix.at[perm].get(mode="promise_in_bounds")
                better = jnp.logical_or(
                    pd2 < bst, jnp.logical_and(pd2 == bst, pix < bix))
                bst = jnp.where(better, pd2, bst)
                bix = jnp.where(better, pix, bix)
            outs.append((bst, bix))
        return outs

    best = jnp.full((L,), BIG, jnp.float32)
    bidx = jnp.zeros((L,), jnp.int32)
    for h in range(0, L, TC_CHUNK):
        for k, (bst, bix) in enumerate(scan_targets(range(h, h + TC_CHUNK))):
            sel = lane == (h + k)
            best = jnp.where(sel, bst, best)
            bidx = jnp.where(sel, bix, bidx)

    matched = best <= RADIUS2
    lrow[0, :] = jnp.where(matched, 1.0, 0.0)
    lrow[1, :] = bidx.astype(jnp.float32)
    pltpu.sync_copy(lrow, sh.at[2 * L + s])
    plsc.subcore_barrier()

    @pl.when(sg == 0)
    def _dedup():
        pltpu.sync_copy(sh.at[pl.ds(2 * L, L)], mb)
        srow = (s // 8) * 8
        tp = jnp.zeros((L,), jnp.int32)
        lane = lax.broadcasted_iota(jnp.int32, (L,), 0)
        masks = [mb[srow + r, 0, :] > 0.5 for r in range(n_groups)]
        idxs = [mb[srow + r, 1, :].astype(jnp.int32) for r in range(n_groups)]
        tags = [jnp.full((L,), r * L, jnp.int32) + lane for r in range(n_groups)]
        for r in range(n_groups):
            plsc.store_scatter(table, [idxs[r]], tags[r], mask=masks[r])
        for r in range(n_groups):
            g = plsc.load_gather(table, [idxs[r]], mask=masks[r])
            won = jnp.logical_and(masks[r], g == tags[r])
            tp = tp + plsc.all_reduce_population_count(won)
        orow[...] = tp.astype(jnp.float32)
        pltpu.sync_copy(orow, out_hbm.at[b])


def kernel(pred, gt):
    B, P, _ = pred.shape
    T = gt.shape[1]
    TPAD = 128
    n_groups = TPAD // L

    pred_t = jnp.transpose(pred, (0, 2, 1))
    gt_pad = jnp.pad(gt, ((0, 0), (0, TPAD - T), (0, 0)), constant_values=-1.0)
    gt_t = jnp.transpose(gt_pad, (0, 2, 1)).reshape(B, 3, n_groups, L)
    gt_t = jnp.transpose(gt_t, (0, 2, 1, 3))

    mesh = plsc.VectorSubcoreMesh(core_axis_name="c", subcore_axis_name="s")
    body = functools.partial(_sc_body, n_p=P, n_groups=n_groups)
    tp_rows = pl.kernel(
        body,
        out_type=jax.ShapeDtypeStruct((B, L), jnp.float32),
        mesh=mesh,
        compiler_params=pltpu.CompilerParams(needs_layout_passes=False),
        scratch_types=[
            pltpu.VMEM((3, P), jnp.float32),
            pltpu.VMEM((3, L), jnp.float32),
            pltpu.VMEM((P,), jnp.int32),
            pltpu.VMEM((2, L), jnp.float32),
            pltpu.VMEM((L, 2, L), jnp.float32),
            pltpu.VMEM((L,), jnp.float32),
            pltpu.VMEM_SHARED((3 * L, 2, L), jnp.float32),
        ],
    )(pred_t, gt_t)

    tp = jnp.sum(tp_rows[:, 0])
    fp = jnp.float32(B * P) - tp
    fn = jnp.float32(B * T) - tp
    prec = (tp + 1e-06) / (tp + 1e-06 + fp + 1e-06)
    rec = (tp + 1e-06) / (tp + fn + 1e-06)
    f1 = 2.0 * prec * rec / (prec + rec)
    return 1.0 - f1


# speedup vs baseline: 1.8862x; 1.8862x over previous
import functools

import jax
import jax.numpy as jnp
from jax import lax
from jax.experimental import pallas as pl
from jax.experimental.pallas import tpu as pltpu
from jax.experimental.pallas import tpu_sc as plsc

RADIUS2 = 25.0
BIG = 1e30
L = 16


def _sc_body(pred_hbm, gt_hbm, out_hbm, pbuf, tbuf, table, lrow, mb, orow,
             sh, *, n_p, n_rows):
    c = lax.axis_index("c")
    s = lax.axis_index("s")

    pltpu.sync_copy(pred_hbm.at[c], pbuf)
    pltpu.sync_copy(gt_hbm.at[c, s], tbuf)

    tcv = tbuf[0, :]
    txv = tbuf[1, :]
    tyv = tbuf[2, :]

    lane = lax.broadcasted_iota(jnp.int32, (L,), 0)
    N_T = 8
    UNROLL = 2

    tb = [jnp.full((L,), t, jnp.int32) for t in range(N_T)]
    txb = [txv.at[i].get(mode="promise_in_bounds") for i in tb]
    tyb = [tyv.at[i].get(mode="promise_in_bounds") for i in tb]
    tcb = [tcv.at[i].get(mode="promise_in_bounds") for i in tb]

    def tile_step(i, carry):
        accs = list(carry)
        for u in range(UNROLL):
            base = (UNROLL * i + u) * L
            pxv = pbuf[1, pl.ds(base, L)]
            pyv = pbuf[2, pl.ds(base, L)]
            pcv = pbuf[0, pl.ds(base, L)]
            idxv = jnp.full((L,), base, jnp.int32) + lane
            for k in range(N_T):
                bst, bix = accs[2 * k], accs[2 * k + 1]
                dx = pxv - txb[k]
                dy = pyv - tyb[k]
                d2 = dx * dx + dy * dy
                key = jnp.where(pcv == tcb[k], d2, BIG)
                lt = key < bst
                accs[2 * k] = jnp.where(lt, key, bst)
                accs[2 * k + 1] = jnp.where(lt, idxv, bix)
        return tuple(accs)

    init = []
    for _ in range(N_T):
        init += [jnp.full((L,), BIG, jnp.float32), jnp.zeros((L,), jnp.int32)]
    accs = lax.fori_loop(0, n_p // (L * UNROLL), tile_step, tuple(init))

    best = jnp.full((L,), BIG, jnp.float32)
    bidx = jnp.zeros((L,), jnp.int32)
    for k in range(N_T):
        bst, bix = accs[2 * k], accs[2 * k + 1]
        for step in (8, 4, 2, 1):
            perm = jnp.bitwise_and(lane + step, L - 1)
            pd2 = bst.at[perm].get(mode="promise_in_bounds")
            pix = bix.at[perm].get(mode="promise_in_bounds")
            better = jnp.logical_or(
                pd2 < bst, jnp.logical_and(pd2 == bst, pix < bix))
            bst = jnp.where(better, pd2, bst)
            bix = jnp.where(better, pix, bix)
        sel = lane == k
        best = jnp.where(sel, bst, best)
        bidx = jnp.where(sel, bix, bidx)

    matched = best <= RADIUS2
    lrow[0, :] = jnp.where(matched, 1.0, 0.0)
    lrow[1, :] = bidx.astype(jnp.float32)
    pltpu.sync_copy(lrow, sh.at[2 * L + s])
    plsc.subcore_barrier()

    @pl.when(s == 0)
    def _dedup():
        pltpu.sync_copy(sh.at[pl.ds(2 * L, L)], mb)
        tp = jnp.zeros((L,), jnp.int32)
        lane2 = lax.broadcasted_iota(jnp.int32, (L,), 0)
        masks = [mb[r, 0, :] > 0.5 for r in range(n_rows)]
        idxs = [mb[r, 1, :].astype(jnp.int32) for r in range(n_rows)]
        tags = [jnp.full((L,), r * L, jnp.int32) + lane2 for r in range(n_rows)]
        for r in range(n_rows):
            plsc.store_scatter(table, [idxs[r]], tags[r], mask=masks[r])
        for r in range(n_rows):
            g = plsc.load_gather(table, [idxs[r]], mask=masks[r])
            won = jnp.logical_and(masks[r], g == tags[r])
            tp = tp + plsc.all_reduce_population_count(won)
        orow[...] = tp.astype(jnp.float32)
        pltpu.sync_copy(orow, out_hbm.at[c])


def _sc_part(pred_t, gt_t, n_p):
    mesh = plsc.VectorSubcoreMesh(core_axis_name="c", subcore_axis_name="s")
    body = functools.partial(_sc_body, n_p=n_p, n_rows=L)
    return pl.kernel(
        body,
        out_type=jax.ShapeDtypeStruct((2, L), jnp.float32),
        mesh=mesh,
        compiler_params=pltpu.CompilerParams(needs_layout_passes=False),
        scratch_types=[
            pltpu.VMEM((3, n_p), jnp.float32),
            pltpu.VMEM((3, L), jnp.float32),
            pltpu.VMEM((n_p,), jnp.int32),
            pltpu.VMEM((2, L), jnp.float32),
            pltpu.VMEM((L, 2, L), jnp.float32),
            pltpu.VMEM((L,), jnp.float32),
            pltpu.VMEM_SHARED((3 * L, 2, L), jnp.float32),
        ],
    )(pred_t, gt_t)


def _tc_body(pred_ref, gt_ref, out_ref, best_ref, bidx_ref, tp_ref, *,
             nb_total, pb, n_b):
    b = pl.program_id(0)
    nb = pl.program_id(1)

    @pl.when(jnp.logical_and(b == 0, nb == 0))
    def _init_tp():
        tp_ref[0, 0] = 0.0

    @pl.when(nb == 0)
    def _init_sample():
        best_ref[...] = jnp.full((1, 128), BIG, jnp.float32)
        bidx_ref[...] = jnp.zeros((1, 128), jnp.int32)

    pc = pred_ref[0, :, 0:1]
    px = pred_ref[0, :, 1:2]
    py = pred_ref[0, :, 2:3]
    tc = gt_ref[0, 0:1, :]
    tx = gt_ref[0, 1:2, :]
    ty = gt_ref[0, 2:3, :]

    dx = px - tx
    dy = py - ty
    d2 = dx * dx + dy * dy
    valid = jnp.logical_and(pc == tc, d2 <= RADIUS2)
    key = jnp.where(valid, d2, BIG)

    blk_min = jnp.min(key, axis=0, keepdims=True)
    rows = jax.lax.broadcasted_iota(jnp.int32, (pb, 128), 0) + nb * pb
    blk_idx = jnp.min(jnp.where(key == blk_min, rows, jnp.int32(2**30)),
                      axis=0, keepdims=True)

    upd = blk_min < best_ref[...]
    best_ref[...] = jnp.where(upd, blk_min, best_ref[...])
    bidx_ref[...] = jnp.where(upd, blk_idx, bidx_ref[...])

    @pl.when(nb == nb_total - 1)
    def _finalize():
        best = best_ref[...]
        matched = best <= RADIUS2
        lane = jax.lax.broadcasted_iota(jnp.int32, (1, 128), 1)
        fkey = jnp.where(matched, bidx_ref[...], -1 - lane).astype(jnp.float32)
        r0 = jax.lax.broadcasted_iota(jnp.int32, (128, 128), 0)
        r1 = jax.lax.broadcasted_iota(jnp.int32, (128, 128), 1)
        ident = (r0 == r1).astype(jnp.float32)
        fkey_col = jax.lax.dot_general(
            ident, fkey, (((1,), (1,)), ((), ())),
            preferred_element_type=jnp.float32)
        eq = fkey_col == fkey
        earlier = r1 < r0
        dup = jnp.max(jnp.logical_and(eq, earlier).astype(jnp.float32),
                      axis=1, keepdims=True)
        n_matched = jnp.sum(matched.astype(jnp.float32))
        n_dup = jnp.sum(dup)
        tp_ref[0, 0] = tp_ref[0, 0] + (n_matched - n_dup)

        @pl.when(b == n_b - 1)
        def _out():
            out_ref[...] = jnp.full((1, 1), tp_ref[0, 0], jnp.float32)


def _tc_part(pred, gt_t, b_off, n_b, n_p):
    PB = 2000
    NB = n_p // PB
    body = functools.partial(_tc_body, nb_total=NB, pb=PB, n_b=n_b)
    return pl.pallas_call(
        body,
        grid=(n_b, NB),
        in_specs=[
            pl.BlockSpec((1, PB, 3), lambda b, nb: (b + b_off, nb, 0)),
            pl.BlockSpec((1, 3, 128), lambda b, nb: (b + b_off, 0, 0)),
        ],
        out_specs=pl.BlockSpec((1, 1), lambda b, nb: (0, 0)),
        out_shape=jax.ShapeDtypeStruct((1, 1), jnp.float32),
        scratch_shapes=[
            pltpu.VMEM((1, 128), jnp.float32),
            pltpu.VMEM((1, 128), jnp.int32),
            pltpu.SMEM((1, 1), jnp.float32),
        ],
    )(pred, gt_t)


def kernel(pred, gt):
    B, P, _ = pred.shape
    T = gt.shape[1]
    TPAD = 128
    B_SC = 2

    gt_pad = jnp.pad(gt, ((0, 0), (0, TPAD - T), (0, 0)), constant_values=-1.0)

    pred_sc = jnp.transpose(pred[:B_SC], (0, 2, 1))
    gt_sc = jnp.pad(gt_pad[:B_SC].reshape(B_SC, L, 8, 3),
                    ((0, 0), (0, 0), (0, 8), (0, 0)), constant_values=-1.0)
    gt_sc = jnp.transpose(gt_sc, (0, 1, 3, 2))

    gt_tc = jnp.transpose(gt_pad, (0, 2, 1))

    tp_sc_rows = _sc_part(pred_sc, gt_sc, P)
    tp_tc = _tc_part(pred, gt_tc, B_SC, B - B_SC, P)

    tp = jnp.sum(tp_sc_rows[:, 0]) + tp_tc[0, 0]
    fp = jnp.float32(B * P) - tp
    fn = jnp.float32(B * T) - tp
    prec = (tp + 1e-06) / (tp + 1e-06 + fp + 1e-06)
    rec = (tp + 1e-06) / (tp + fn + 1e-06)
    f1 = 2.0 * prec * rec / (prec + rec)
    return 1.0 - f1


# speedup vs baseline: 1.8906x; 1.0023x over previous
import functools

import jax
import jax.numpy as jnp
from jax import lax
from jax.experimental import pallas as pl
from jax.experimental.pallas import tpu as pltpu
from jax.experimental.pallas import tpu_sc as plsc

RADIUS2 = 25.0
BIG = 1e30
L = 16


def _sc_body(pred_hbm, gt_hbm, out_hbm, pbuf, tbuf, table, lrow, mb, orow,
             sh, *, n_p, n_rows):
    c = lax.axis_index("c")
    s = lax.axis_index("s")

    pltpu.sync_copy(pred_hbm.at[c], pbuf)
    pltpu.sync_copy(gt_hbm.at[c, s], tbuf)

    tcv = tbuf[0, :]
    txv = tbuf[1, :]
    tyv = tbuf[2, :]

    lane = lax.broadcasted_iota(jnp.int32, (L,), 0)
    N_T = 8
    UNROLL = 2

    tb = [jnp.full((L,), t, jnp.int32) for t in range(N_T)]
    txb = [txv.at[i].get(mode="promise_in_bounds") for i in tb]
    tyb = [tyv.at[i].get(mode="promise_in_bounds") for i in tb]
    tcb = [tcv.at[i].get(mode="promise_in_bounds") for i in tb]

    def tile_step(i, carry):
        accs = list(carry)
        for u in range(UNROLL):
            base = (UNROLL * i + u) * L
            pxv = pbuf[1, pl.ds(base, L)]
            pyv = pbuf[2, pl.ds(base, L)]
            pcv = pbuf[0, pl.ds(base, L)]
            idxv = jnp.full((L,), base, jnp.int32) + lane
            for k in range(N_T):
                bst, bix = accs[2 * k], accs[2 * k + 1]
                dx = pxv - txb[k]
                dy = pyv - tyb[k]
                d2 = dx * dx + dy * dy
                key = jnp.where(pcv == tcb[k], d2, BIG)
                lt = key < bst
                accs[2 * k] = jnp.where(lt, key, bst)
                accs[2 * k + 1] = jnp.where(lt, idxv, bix)
        return tuple(accs)

    init = []
    for _ in range(N_T):
        init += [jnp.full((L,), BIG, jnp.float32), jnp.zeros((L,), jnp.int32)]
    accs = lax.fori_loop(0, n_p // (L * UNROLL), tile_step, tuple(init))

    best = jnp.full((L,), BIG, jnp.float32)
    bidx = jnp.zeros((L,), jnp.int32)
    for k in range(N_T):
        bst, bix = accs[2 * k], accs[2 * k + 1]
        for step in (8, 4, 2, 1):
            perm = jnp.bitwise_and(lane + step, L - 1)
            pd2 = bst.at[perm].get(mode="promise_in_bounds")
            pix = bix.at[perm].get(mode="promise_in_bounds")
            better = jnp.logical_or(
                pd2 < bst, jnp.logical_and(pd2 == bst, pix < bix))
            bst = jnp.where(better, pd2, bst)
            bix = jnp.where(better, pix, bix)
        sel = lane == k
        best = jnp.where(sel, bst, best)
        bidx = jnp.where(sel, bix, bidx)

    matched = best <= RADIUS2
    lrow[0, :] = jnp.where(matched, 1.0, 0.0)
    lrow[1, :] = bidx.astype(jnp.float32)
    pltpu.sync_copy(lrow, sh.at[2 * L + s])
    plsc.subcore_barrier()

    @pl.when(s == 0)
    def _dedup():
        pltpu.sync_copy(sh.at[pl.ds(2 * L, L)], mb)
        tp = jnp.zeros((L,), jnp.int32)
        lane2 = lax.broadcasted_iota(jnp.int32, (L,), 0)
        masks = [mb[r, 0, :] > 0.5 for r in range(n_rows)]
        idxs = [mb[r, 1, :].astype(jnp.int32) for r in range(n_rows)]
        tags = [jnp.full((L,), r * L, jnp.int32) + lane2 for r in range(n_rows)]
        for r in range(n_rows):
            plsc.store_scatter(table, [idxs[r]], tags[r], mask=masks[r])
        for r in range(n_rows):
            g = plsc.load_gather(table, [idxs[r]], mask=masks[r])
            won = jnp.logical_and(masks[r], g == tags[r])
            tp = tp + plsc.all_reduce_population_count(won)
        orow[...] = tp.astype(jnp.float32)
        pltpu.sync_copy(orow, out_hbm.at[c])


def _sc_part(pred_t, gt_t, n_p):
    mesh = plsc.VectorSubcoreMesh(core_axis_name="c", subcore_axis_name="s")
    body = functools.partial(_sc_body, n_p=n_p, n_rows=L)
    return pl.kernel(
        body,
        out_type=jax.ShapeDtypeStruct((2, L), jnp.float32),
        mesh=mesh,
        compiler_params=pltpu.CompilerParams(needs_layout_passes=False),
        scratch_types=[
            pltpu.VMEM((3, n_p), jnp.float32),
            pltpu.VMEM((3, L), jnp.float32),
            pltpu.VMEM((n_p,), jnp.int32),
            pltpu.VMEM((2, L), jnp.float32),
            pltpu.VMEM((L, 2, L), jnp.float32),
            pltpu.VMEM((L,), jnp.float32),
            pltpu.VMEM_SHARED((3 * L, 2, L), jnp.float32),
        ],
    )(pred_t, gt_t)


def _tc_body(pred_ref, gt_ref, out_ref, best_ref, bidx_ref, tp_ref, *,
             nb_total, pb, n_b):
    b = pl.program_id(0)
    nb = pl.program_id(1)

    @pl.when(jnp.logical_and(b == 0, nb == 0))
    def _init_tp():
        tp_ref[0, 0] = 0.0

    @pl.when(nb == 0)
    def _init_sample():
        best_ref[...] = jnp.full((1, 128), BIG, jnp.float32)
        bidx_ref[...] = jnp.zeros((1, 128), jnp.int32)

    pc = pred_ref[0, :, 0:1]
    px = pred_ref[0, :, 1:2]
    py = pred_ref[0, :, 2:3]
    tc = gt_ref[0, 0:1, :]
    tx = gt_ref[0, 1:2, :]
    ty = gt_ref[0, 2:3, :]

    dx = px - tx
    dy = py - ty
    d2 = dx * dx + dy * dy
    valid = jnp.logical_and(pc == tc, d2 <= RADIUS2)
    key = jnp.where(valid, d2, BIG)

    blk_min = jnp.min(key, axis=0, keepdims=True)
    rows = jax.lax.broadcasted_iota(jnp.int32, (pb, 128), 0) + nb * pb
    blk_idx = jnp.min(jnp.where(key == blk_min, rows, jnp.int32(2**30)),
                      axis=0, keepdims=True)

    upd = blk_min < best_ref[...]
    best_ref[...] = jnp.where(upd, blk_min, best_ref[...])
    bidx_ref[...] = jnp.where(upd, blk_idx, bidx_ref[...])

    @pl.when(nb == nb_total - 1)
    def _finalize():
        best = best_ref[...]
        matched = best <= RADIUS2
        lane = jax.lax.broadcasted_iota(jnp.int32, (1, 128), 1)
        fkey = jnp.where(matched, bidx_ref[...], -1 - lane).astype(jnp.float32)
        r0 = jax.lax.broadcasted_iota(jnp.int32, (128, 128), 0)
        r1 = jax.lax.broadcasted_iota(jnp.int32, (128, 128), 1)
        ident = (r0 == r1).astype(jnp.float32)
        fkey_col = jax.lax.dot_general(
            ident, fkey, (((1,), (1,)), ((), ())),
            preferred_element_type=jnp.float32)
        eq = fkey_col == fkey
        earlier = r1 < r0
        dup = jnp.max(jnp.logical_and(eq, earlier).astype(jnp.float32),
                      axis=1, keepdims=True)
        n_matched = jnp.sum(matched.astype(jnp.float32))
        n_dup = jnp.sum(dup)
        tp_ref[0, 0] = tp_ref[0, 0] + (n_matched - n_dup)

        @pl.when(b == n_b - 1)
        def _out():
            out_ref[...] = jnp.full((1, 1), tp_ref[0, 0], jnp.float32)


def _tc_part(pred, gt_t, b_off, n_b, n_p):
    PB = 2000
    NB = n_p // PB
    body = functools.partial(_tc_body, nb_total=NB, pb=PB, n_b=n_b)
    return pl.pallas_call(
        body,
        grid=(n_b, NB),
        in_specs=[
            pl.BlockSpec((1, PB, 3), lambda b, nb: (b + b_off, nb, 0)),
            pl.BlockSpec((1, 3, 128), lambda b, nb: (b + b_off, 0, 0)),
        ],
        out_specs=pl.BlockSpec((1, 1), lambda b, nb: (0, 0)),
        out_shape=jax.ShapeDtypeStruct((1, 1), jnp.float32),
        scratch_shapes=[
            pltpu.VMEM((1, 128), jnp.float32),
            pltpu.VMEM((1, 128), jnp.int32),
            pltpu.SMEM((1, 1), jnp.float32),
        ],
    )(pred, gt_t)


def kernel(pred, gt):
    B, P, _ = pred.shape
    T = gt.shape[1]
    TPAD = 128
    B_SC = 2

    gt_pad = jnp.pad(gt, ((0, 0), (0, TPAD - T), (0, 0)), constant_values=-1.0)

    pred_sc = jnp.transpose(pred[:B_SC], (0, 2, 1))
    gt_sc = jnp.pad(gt_pad[:B_SC].reshape(B_SC, L, 8, 3),
                    ((0, 0), (0, 0), (0, 8), (0, 0)), constant_values=-1.0)
    gt_sc = jnp.transpose(gt_sc, (0, 1, 3, 2))

    gt_tc = jnp.transpose(gt_pad, (0, 2, 1))

    tp_tc = _tc_part(pred, gt_tc, B_SC, B - B_SC, P)
    tp_sc_rows = _sc_part(pred_sc, gt_sc, P)

    tp = jnp.sum(tp_sc_rows[:, 0]) + tp_tc[0, 0]
    fp = jnp.float32(B * P) - tp
    fn = jnp.float32(B * T) - tp
    prec = (tp + 1e-06) / (tp + 1e-06 + fp + 1e-06)
    rec = (tp + 1e-06) / (tp + fn + 1e-06)
    f1 = 2.0 * prec * rec / (prec + rec)
    return 1.0 - f1


# speedup vs baseline: 2.0088x; 1.0626x over previous
import functools

import jax
import jax.numpy as jnp
from jax import lax
from jax.experimental import pallas as pl
from jax.experimental.pallas import tpu as pltpu
from jax.experimental.pallas import tpu_sc as plsc

RADIUS2 = 25.0
BIG = 1e30
L = 16


def _sc_body(pred_hbm, gt_hbm, out_hbm, pbuf, tbuf, table, lrow, mb, orow,
             sh, *, n_p, n_groups):
    c = lax.axis_index("c")
    s = lax.axis_index("s")
    b = 2 * c + s // 8
    sg = s % 8

    pltpu.sync_copy(pred_hbm.at[b], pbuf)
    pltpu.sync_copy(gt_hbm.at[b, sg], tbuf)

    tcv = tbuf[0, :]
    txv = tbuf[1, :]
    tyv = tbuf[2, :]

    lane = lax.broadcasted_iota(jnp.int32, (L,), 0)

    TC_CHUNK = 8
    UNROLL = 2

    def scan_targets(tslice):
        tb = [jnp.full((L,), t, jnp.int32) for t in tslice]
        txb = [txv.at[i].get(mode="promise_in_bounds") for i in tb]
        tyb = [tyv.at[i].get(mode="promise_in_bounds") for i in tb]
        tcb = [tcv.at[i].get(mode="promise_in_bounds") for i in tb]

        def tile_step(i, carry):
            accs = list(carry)
            for u in range(UNROLL):
                base = (UNROLL * i + u) * L
                pxv = pbuf[1, pl.ds(base, L)]
                pyv = pbuf[2, pl.ds(base, L)]
                pcv = pbuf[0, pl.ds(base, L)]
                idxv = jnp.full((L,), base, jnp.int32) + lane
                for k in range(len(tslice)):
                    bst, bix = accs[2 * k], accs[2 * k + 1]
                    dx = pxv - txb[k]
                    dy = pyv - tyb[k]
                    d2 = dx * dx + dy * dy
                    key = jnp.where(pcv == tcb[k], d2, BIG)
                    lt = key < bst
                    accs[2 * k] = jnp.where(lt, key, bst)
                    accs[2 * k + 1] = jnp.where(lt, idxv, bix)
            return tuple(accs)

        init = []
        for _ in tslice:
            init += [jnp.full((L,), BIG, jnp.float32), jnp.zeros((L,), jnp.int32)]
        accs = lax.fori_loop(0, n_p // (L * UNROLL), tile_step, tuple(init))

        outs = []
        for k in range(len(tslice)):
            bst, bix = accs[2 * k], accs[2 * k + 1]
            for step in (8, 4, 2, 1):
                perm = jnp.bitwise_and(lane + step, L - 1)
                pd2 = bst.at[perm].get(mode="promise_in_bounds")
                pix = bix.at[perm].get(mode="promise_in_bounds")
                better = jnp.logical_or(
                    pd2 < bst, jnp.logical_and(pd2 == bst, pix < bix))
                bst = jnp.where(better, pd2, bst)
                bix = jnp.where(better, pix, bix)
            outs.append((bst, bix))
        return outs

    best = jnp.full((L,), BIG, jnp.float32)
    bidx = jnp.zeros((L,), jnp.int32)
    for h in range(0, L, TC_CHUNK):
        for k, (bst, bix) in enumerate(scan_targets(range(h, h + TC_CHUNK))):
            sel = lane == (h + k)
            best = jnp.where(sel, bst, best)
            bidx = jnp.where(sel, bix, bidx)

    matched = best <= RADIUS2
    lrow[0, :] = jnp.where(matched, 1.0, 0.0)
    lrow[1, :] = bidx.astype(jnp.float32)
    pltpu.sync_copy(lrow, sh.at[2 * L + s])
    plsc.subcore_barrier()

    @pl.when(sg == 0)
    def _dedup():
        pltpu.sync_copy(sh.at[pl.ds(2 * L, L)], mb)
        srow = (s // 8) * 8
        tp = jnp.zeros((L,), jnp.int32)
        lane = lax.broadcasted_iota(jnp.int32, (L,), 0)
        masks = [mb[srow + r, 0, :] > 0.5 for r in range(n_groups)]
        idxs = [mb[srow + r, 1, :].astype(jnp.int32) for r in range(n_groups)]
        tags = [jnp.full((L,), r * L, jnp.int32) + lane for r in range(n_groups)]
        for r in range(n_groups):
            plsc.store_scatter(table, [idxs[r]], tags[r], mask=masks[r])
        for r in range(n_groups):
            g = plsc.load_gather(table, [idxs[r]], mask=masks[r])
            won = jnp.logical_and(masks[r], g == tags[r])
            tp = tp + plsc.all_reduce_population_count(won)
        orow[...] = tp.astype(jnp.float32)
        pltpu.sync_copy(orow, out_hbm.at[b])


def kernel(pred, gt):
    B, P, _ = pred.shape
    T = gt.shape[1]
    TPAD = 128
    n_groups = TPAD // L

    pred_t = jnp.transpose(pred, (0, 2, 1))
    gt_pad = jnp.pad(gt, ((0, 0), (0, TPAD - T), (0, 0)), constant_values=-1.0)
    gt_t = jnp.transpose(gt_pad, (0, 2, 1)).reshape(B, 3, n_groups, L)
    gt_t = jnp.transpose(gt_t, (0, 2, 1, 3))

    mesh = plsc.VectorSubcoreMesh(core_axis_name="c", subcore_axis_name="s")
    body = functools.partial(_sc_body, n_p=P, n_groups=n_groups)
    tp_rows = pl.kernel(
        body,
        out_type=jax.ShapeDtypeStruct((B, L), jnp.float32),
        mesh=mesh,
        compiler_params=pltpu.CompilerParams(needs_layout_passes=False),
        scratch_types=[
            pltpu.VMEM((3, P), jnp.float32),
            pltpu.VMEM((3, L), jnp.float32),
            pltpu.VMEM((P,), jnp.int32),
            pltpu.VMEM((2, L), jnp.float32),
            pltpu.VMEM((L, 2, L), jnp.float32),
            pltpu.VMEM((L,), jnp.float32),
            pltpu.VMEM_SHARED((3 * L, 2, L), jnp.float32),
        ],
    )(pred_t, gt_t)

    tp = jnp.sum(tp_rows[:, 0])
    fp = jnp.float32(B * P) - tp
    fn = jnp.float32(B * T) - tp
    prec = (tp + 1e-06) / (tp + 1e-06 + fp + 1e-06)
    rec = (tp + 1e-06) / (tp + fn + 1e-06)
    f1 = 2.0 * prec * rec / (prec + rec)
    return 1.0 - f1
